# baseline (speedup 1.0000x reference)
import jax
import jax.numpy as jnp
from jax.experimental import pallas as pl
from jax.experimental.pallas import tpu as pltpu

N = 2048
INS = 512
CLASSES = 40
H1 = 8
O1 = 8
HD = H1 * O1
LEAK = 0.2
BR = 512
NB2 = N // BR
RG = BR // 8
FW = 72
FW2 = 48


def _fused_kernel(x_ref, w_ref, a1sT_ref, a1dT_ref, adj_ref,
                  w2_ref, a2sT_ref, a2dT_ref, out_ref,
                  adj_s, h1_s, haug_s, es_s, fs2_s, ed_s, fd2_s,
                  faug_s, es2_s, fs22_s, ed2_s, fd22_s, w1r_s, amat_s):
    t = pl.program_id(0)

    @pl.when(t == 0)
    def _l1_prologue():
        amat_s[...] = jnp.zeros((HD, 2 * H1), jnp.float32)
        for h in range(H1):
            w1r_s[:, h * O1:(h + 1) * O1] = w_ref[h]
            amat_s[pl.ds(h * O1, O1), h:h + 1] = a1sT_ref[:, h:h + 1]
            amat_s[pl.ds(h * O1, O1), H1 + h:H1 + h + 1] = a1dT_ref[:, h:h + 1]
        hall = jnp.dot(x_ref[...], w1r_s[...],
                       preferred_element_type=jnp.float32)
        fboth = jnp.dot(hall, amat_s[...],
                        preferred_element_type=jnp.float32)
        fsrc = fboth[:, 0:H1]
        fdst = fboth[:, H1:2 * H1]
        alpha = jnp.max(fsrc, axis=0, keepdims=True)
        es_s[...] = jnp.exp(fsrc - alpha).reshape(N // 8, 8, H1)
        fs2_s[...] = jnp.exp(LEAK * fsrc - alpha).reshape(N // 8, 8, H1)
        fdT = jnp.transpose(fdst)
        beta = jnp.max(fdT, axis=1, keepdims=True)
        edT = jnp.exp(fdT - beta)
        fd2T = jnp.exp(LEAK * fdT - beta)
        for h in range(H1):
            ed_s[h, :, :] = jnp.broadcast_to(edT[h:h + 1, :], (8, N))
            fd2_s[h, :, :] = jnp.broadcast_to(fd2T[h:h + 1, :], (8, N))
        haug_s[:, 0:HD] = hall
        haug_s[:, HD:HD + 1] = jnp.ones((N, 1), jnp.float32)
        haug_s[:, HD + 1:FW] = jnp.zeros((N, FW - HD - 1), jnp.float32)

    @pl.when(t < NB2)
    def _l1_body():
        adjb = adj_ref[...]
        adj_s[pl.ds(t * BR, BR), :] = adjb
        adj3 = adjb.reshape(RG, 8, N)
        haug = haug_s[...]
        for h in range(H1):
            es = es_s[pl.ds(t * RG, RG), :, h:h + 1]
            fs2 = fs2_s[pl.ds(t * RG, RG), :, h:h + 1]
            u = es * ed_s[h:h + 1, :, :]
            v = fs2 * fd2_s[h:h + 1, :, :]
            p = (jnp.maximum(u, v) * adj3).reshape(BR, N)
            agg = jnp.dot(p, haug,
                          preferred_element_type=jnp.float32)
            o = agg[:, h * O1:(h + 1) * O1] / agg[:, HD:HD + 1]
            h1_s[pl.ds(t * BR, BR), h * O1:(h + 1) * O1] = jnp.where(
                o > 0, o, jnp.exp(o) - 1.0)

    @pl.when(t == NB2)
    def _l2_prologue():
        h1 = h1_s[...]
        feat = jnp.dot(h1, w2_ref[0],
                       preferred_element_type=jnp.float32)
        faug_s[:, 0:CLASSES] = feat
        faug_s[:, CLASSES:CLASSES + 1] = jnp.ones((N, 1), jnp.float32)
        faug_s[:, CLASSES + 1:FW2] = jnp.zeros((N, FW2 - CLASSES - 1),
                                               jnp.float32)
        fsrc = jnp.dot(feat, a2sT_ref[...],
                       preferred_element_type=jnp.float32)
        fdst = jnp.dot(feat, a2dT_ref[...],
                       preferred_element_type=jnp.float32)
        alpha = jnp.max(fsrc)
        es2_s[...] = jnp.exp(fsrc - alpha).reshape(N // 8, 8, 1)
        fs22_s[...] = jnp.exp(LEAK * fsrc - alpha).reshape(N // 8, 8, 1)
        fdT = jnp.transpose(fdst)
        beta = jnp.max(fdT)
        ed2_s[...] = jnp.broadcast_to(jnp.exp(fdT - beta), (8, N))[None]
        fd22_s[...] = jnp.broadcast_to(jnp.exp(LEAK * fdT - beta),
                                       (8, N))[None]

    @pl.when(t >= NB2)
    def _l2_body():
        j = t - NB2
        adj3 = adj_s[pl.ds(j * BR, BR), :].reshape(RG, 8, N)
        es = es2_s[pl.ds(j * RG, RG), :, :]
        fs2 = fs22_s[pl.ds(j * RG, RG), :, :]
        u = es * ed2_s[...]
        v = fs2 * fd22_s[...]
        p = (jnp.maximum(u, v) * adj3).reshape(BR, N)
        agg = jnp.dot(p, faug_s[...],
                      preferred_element_type=jnp.float32)
        z = agg[:, 0:CLASSES] / agg[:, CLASSES:CLASSES + 1]
        m2 = jnp.max(z, axis=1, keepdims=True)
        lse = m2 + jnp.log(jnp.sum(jnp.exp(z - m2), axis=1, keepdims=True))
        out_ref[...] = z - lse


def kernel(x, adj, W1, a1_src, a1_dst, W2, a2_src, a2_dst):
    a1sT = jnp.transpose(a1_src)
    a1dT = jnp.transpose(a1_dst)
    a2sT = jnp.transpose(a2_src)
    a2dT = jnp.transpose(a2_dst)

    out = pl.pallas_call(
        _fused_kernel,
        grid=(2 * NB2,),
        in_specs=[
            pl.BlockSpec((N, INS), lambda t: (0, 0)),
            pl.BlockSpec((H1, INS, O1), lambda t: (0, 0, 0)),
            pl.BlockSpec((O1, H1), lambda t: (0, 0)),
            pl.BlockSpec((O1, H1), lambda t: (0, 0)),
            pl.BlockSpec((BR, N), lambda t: (jnp.minimum(t, NB2 - 1), 0)),
            pl.BlockSpec((1, HD, CLASSES), lambda t: (0, 0, 0)),
            pl.BlockSpec((CLASSES, 1), lambda t: (0, 0)),
            pl.BlockSpec((CLASSES, 1), lambda t: (0, 0)),
        ],
        out_specs=pl.BlockSpec(
            (BR, CLASSES), lambda t: (jnp.where(t < NB2, t, t - NB2), 0)),
        out_shape=jax.ShapeDtypeStruct((N, CLASSES), jnp.float32),
        scratch_shapes=[
            pltpu.VMEM((N, N), jnp.float32),
            pltpu.VMEM((N, HD), jnp.float32),
            pltpu.VMEM((N, FW), jnp.float32),
            pltpu.VMEM((N // 8, 8, H1), jnp.float32),
            pltpu.VMEM((N // 8, 8, H1), jnp.float32),
            pltpu.VMEM((H1, 8, N), jnp.float32),
            pltpu.VMEM((H1, 8, N), jnp.float32),
            pltpu.VMEM((N, FW2), jnp.float32),
            pltpu.VMEM((N // 8, 8, 1), jnp.float32),
            pltpu.VMEM((N // 8, 8, 1), jnp.float32),
            pltpu.VMEM((1, 8, N), jnp.float32),
            pltpu.VMEM((1, 8, N), jnp.float32),
            pltpu.VMEM((INS, HD), jnp.float32),
            pltpu.VMEM((HD, 2 * H1), jnp.float32),
        ],
    )(x, W1, a1sT, a1dT, adj, W2, a2sT, a2dT)
    return out


# speedup vs baseline: 1.0122x; 1.0122x over previous
import jax
import jax.numpy as jnp
from jax.experimental import pallas as pl
from jax.experimental.pallas import tpu as pltpu

N = 2048
INS = 512
CLASSES = 40
H1 = 8
O1 = 8
HD = H1 * O1
LEAK = 0.2
BR = 512
NB2 = N // BR
RG = BR // 8
FW = 72
FW2 = 48


def _fused_kernel(x_ref, w_ref, asrc_ref, adst_ref, adj_ref,
                  w2_ref, vs_ref, vd_ref, out_ref,
                  h1_s, haug_s, es_s, fs2_s, ed_s, fd2_s,
                  faug_s, es2_s, fs22_s, ed2_s, fd22_s):
    t = pl.program_id(0)

    @pl.when(t == 0)
    def _l1_prologue():
        hall = jnp.dot(x_ref[...], w_ref[...],
                       preferred_element_type=jnp.float32)
        fsrc = jnp.dot(hall, asrc_ref[...],
                       preferred_element_type=jnp.float32)
        fdst = jnp.dot(hall, adst_ref[...],
                       preferred_element_type=jnp.float32)
        alpha = jnp.max(fsrc, axis=0, keepdims=True)
        es_s[...] = jnp.exp(fsrc - alpha).reshape(N // 8, 8, H1)
        fs2_s[...] = jnp.exp(LEAK * fsrc - alpha).reshape(N // 8, 8, H1)
        fdT = jnp.transpose(fdst)
        beta = jnp.max(fdT, axis=1, keepdims=True)
        edT = jnp.exp(fdT - beta)
        fd2T = jnp.exp(LEAK * fdT - beta)
        for h in range(H1):
            ed_s[h, :, :] = jnp.broadcast_to(edT[h:h + 1, :], (8, N))
            fd2_s[h, :, :] = jnp.broadcast_to(fd2T[h:h + 1, :], (8, N))
        haug_s[:, 0:HD] = hall
        haug_s[:, HD:HD + 1] = jnp.ones((N, 1), jnp.float32)
        haug_s[:, HD + 1:FW] = jnp.zeros((N, FW - HD - 1), jnp.float32)

    @pl.when(t < NB2)
    def _l1_body():
        adj3 = adj_ref[...].reshape(RG, 8, N)
        haug = haug_s[...]
        for h in range(H1):
            es = es_s[pl.ds(t * RG, RG), :, h:h + 1]
            fs2 = fs2_s[pl.ds(t * RG, RG), :, h:h + 1]
            u = es * ed_s[h:h + 1, :, :]
            v = fs2 * fd2_s[h:h + 1, :, :]
            p = (jnp.maximum(u, v) * adj3).reshape(BR, N)
            agg = jnp.dot(p, haug,
                          preferred_element_type=jnp.float32)
            o = agg[:, h * O1:(h + 1) * O1] / agg[:, HD:HD + 1]
            h1_s[pl.ds(t * BR, BR), h * O1:(h + 1) * O1] = jnp.where(
                o > 0, o, jnp.exp(o) - 1.0)

    @pl.when(t == NB2 - 1)
    def _l2_prologue():
        h1 = h1_s[...]
        faug_s[:, 0:CLASSES] = jnp.dot(h1, w2_ref[...],
                                       preferred_element_type=jnp.float32)
        faug_s[:, CLASSES:CLASSES + 1] = jnp.ones((N, 1), jnp.float32)
        faug_s[:, CLASSES + 1:FW2] = jnp.zeros((N, FW2 - CLASSES - 1),
                                               jnp.float32)
        fsrc = jnp.dot(h1, vs_ref[...],
                       preferred_element_type=jnp.float32)
        fdst = jnp.dot(h1, vd_ref[...],
                       preferred_element_type=jnp.float32)
        alpha = jnp.max(fsrc)
        es2_s[...] = jnp.exp(fsrc - alpha).reshape(N // 8, 8, 1)
        fs22_s[...] = jnp.exp(LEAK * fsrc - alpha).reshape(N // 8, 8, 1)
        fdT = jnp.transpose(fdst)
        beta = jnp.max(fdT)
        ed2_s[...] = jnp.broadcast_to(jnp.exp(fdT - beta), (8, N))[None]
        fd22_s[...] = jnp.broadcast_to(jnp.exp(LEAK * fdT - beta),
                                       (8, N))[None]

    @pl.when(t >= NB2)
    def _l2_body():
        j = 2 * NB2 - 1 - t
        adj3 = adj_ref[...].reshape(RG, 8, N)
        es = es2_s[pl.ds(j * RG, RG), :, :]
        fs2 = fs22_s[pl.ds(j * RG, RG), :, :]
        u = es * ed2_s[...]
        v = fs2 * fd22_s[...]
        p = (jnp.maximum(u, v) * adj3).reshape(BR, N)
        agg = jnp.dot(p, faug_s[...],
                      preferred_element_type=jnp.float32)
        z = agg[:, 0:CLASSES] / agg[:, CLASSES:CLASSES + 1]
        m2 = jnp.max(z, axis=1, keepdims=True)
        lse = m2 + jnp.log(jnp.sum(jnp.exp(z - m2), axis=1, keepdims=True))
        out_ref[...] = z - lse


def kernel(x, adj, W1, a1_src, a1_dst, W2, a2_src, a2_dst):
    W1r = jnp.transpose(W1, (1, 0, 2)).reshape(INS, HD)
    eye = jnp.eye(H1, dtype=jnp.float32)
    Asrc = (eye[:, None, :] * a1_src[:, :, None]).reshape(HD, H1)
    Adst = (eye[:, None, :] * a1_dst[:, :, None]).reshape(HD, H1)
    W2r = W2[0]
    vs = jnp.dot(W2r, a2_src[0])[:, None]
    vd = jnp.dot(W2r, a2_dst[0])[:, None]

    out = pl.pallas_call(
        _fused_kernel,
        grid=(2 * NB2,),
        in_specs=[
            pl.BlockSpec((N, INS), lambda t: (0, 0)),
            pl.BlockSpec((INS, HD), lambda t: (0, 0)),
            pl.BlockSpec((HD, H1), lambda t: (0, 0)),
            pl.BlockSpec((HD, H1), lambda t: (0, 0)),
            pl.BlockSpec((BR, N),
                         lambda t: (jnp.where(t < NB2, t, 2 * NB2 - 1 - t),
                                    0)),
            pl.BlockSpec((HD, CLASSES), lambda t: (0, 0)),
            pl.BlockSpec((HD, 1), lambda t: (0, 0)),
            pl.BlockSpec((HD, 1), lambda t: (0, 0)),
        ],
        out_specs=pl.BlockSpec(
            (BR, CLASSES),
            lambda t: (jnp.where(t < NB2, t, 2 * NB2 - 1 - t), 0)),
        out_shape=jax.ShapeDtypeStruct((N, CLASSES), jnp.float32),
        scratch_shapes=[
            pltpu.VMEM((N, HD), jnp.float32),
            pltpu.VMEM((N, FW), jnp.float32),
            pltpu.VMEM((N // 8, 8, H1), jnp.float32),
            pltpu.VMEM((N // 8, 8, H1), jnp.float32),
            pltpu.VMEM((H1, 8, N), jnp.float32),
            pltpu.VMEM((H1, 8, N), jnp.float32),
            pltpu.VMEM((N, FW2), jnp.float32),
            pltpu.VMEM((N // 8, 8, 1), jnp.float32),
            pltpu.VMEM((N // 8, 8, 1), jnp.float32),
            pltpu.VMEM((1, 8, N), jnp.float32),
            pltpu.VMEM((1, 8, N), jnp.float32),
        ],
    )(x, W1r, Asrc, Adst, adj, W2r, vs, vd)
    return out


# speedup vs baseline: 1.0431x; 1.0304x over previous
import jax
import jax.numpy as jnp
from jax.experimental import pallas as pl
from jax.experimental.pallas import tpu as pltpu

N = 2048
INS = 512
CLASSES = 40
H1 = 8
O1 = 8
HD = H1 * O1
LEAK = 0.2
BR = 512
NB2 = N // BR
RG = BR // 8
FW = 72
FW2 = 48


def _fused_kernel(x_ref, w_ref, asrc_ref, adst_ref, adj_ref,
                  w2_ref, vs_ref, vd_ref, out_ref,
                  adj_s, h1_s, haug_s, es_s, fs2_s, ed_s, fd2_s,
                  faug_s, es2_s, fs22_s, ed2_s, fd22_s):
    t = pl.program_id(0)

    @pl.when(t == 0)
    def _l1_prologue():
        hall = jnp.dot(x_ref[...], w_ref[...],
                       preferred_element_type=jnp.float32)
        fsrc = jnp.dot(hall, asrc_ref[...],
                       preferred_element_type=jnp.float32)
        fdst = jnp.dot(hall, adst_ref[...],
                       preferred_element_type=jnp.float32)
        alpha = jnp.max(fsrc, axis=0, keepdims=True)
        es_s[...] = jnp.exp(fsrc - alpha).reshape(N // 8, 8, H1)
        fs2_s[...] = jnp.exp(LEAK * fsrc - alpha).reshape(N // 8, 8, H1)
        fdT = jnp.transpose(fdst)
        beta = jnp.max(fdT, axis=1, keepdims=True)
        edT = jnp.exp(fdT - beta)
        fd2T = jnp.exp(LEAK * fdT - beta)
        for h in range(H1):
            ed_s[h, :, :] = jnp.broadcast_to(edT[h:h + 1, :], (8, N))
            fd2_s[h, :, :] = jnp.broadcast_to(fd2T[h:h + 1, :], (8, N))
        haug_s[:, 0:HD] = hall
        haug_s[:, HD:HD + 1] = jnp.ones((N, 1), jnp.float32)
        haug_s[:, HD + 1:FW] = jnp.zeros((N, FW - HD - 1), jnp.float32)

    @pl.when(t < NB2)
    def _l1_body():
        adjb = adj_ref[...]
        adj_s[pl.ds(t * BR, BR), :] = adjb
        adj3 = adjb.reshape(RG, 8, N)
        haug = haug_s[...]
        for h in range(H1):
            es = es_s[pl.ds(t * RG, RG), :, h:h + 1]
            fs2 = fs2_s[pl.ds(t * RG, RG), :, h:h + 1]
            u = es * ed_s[h:h + 1, :, :]
            v = fs2 * fd2_s[h:h + 1, :, :]
            p = (jnp.maximum(u, v) * adj3).reshape(BR, N)
            agg = jnp.dot(p, haug,
                          preferred_element_type=jnp.float32)
            o = agg[:, h * O1:(h + 1) * O1] / agg[:, HD:HD + 1]
            h1_s[pl.ds(t * BR, BR), h * O1:(h + 1) * O1] = jnp.where(
                o > 0, o, jnp.exp(o) - 1.0)

    @pl.when(t == NB2)
    def _l2_prologue():
        h1 = h1_s[...]
        faug_s[:, 0:CLASSES] = jnp.dot(h1, w2_ref[...],
                                       preferred_element_type=jnp.float32)
        faug_s[:, CLASSES:CLASSES + 1] = jnp.ones((N, 1), jnp.float32)
        faug_s[:, CLASSES + 1:FW2] = jnp.zeros((N, FW2 - CLASSES - 1),
                                               jnp.float32)
        fsrc = jnp.dot(h1, vs_ref[...],
                       preferred_element_type=jnp.float32)
        fdst = jnp.dot(h1, vd_ref[...],
                       preferred_element_type=jnp.float32)
        alpha = jnp.max(fsrc)
        es2_s[...] = jnp.exp(fsrc - alpha).reshape(N // 8, 8, 1)
        fs22_s[...] = jnp.exp(LEAK * fsrc - alpha).reshape(N // 8, 8, 1)
        fdT = jnp.transpose(fdst)
        beta = jnp.max(fdT)
        ed2_s[...] = jnp.broadcast_to(jnp.exp(fdT - beta), (8, N))[None]
        fd22_s[...] = jnp.broadcast_to(jnp.exp(LEAK * fdT - beta),
                                       (8, N))[None]

    @pl.when(t >= NB2)
    def _l2_body():
        j = t - NB2
        adj3 = adj_s[pl.ds(j * BR, BR), :].reshape(RG, 8, N)
        es = es2_s[pl.ds(j * RG, RG), :, :]
        fs2 = fs22_s[pl.ds(j * RG, RG), :, :]
        u = es * ed2_s[...]
        v = fs2 * fd22_s[...]
        p = (jnp.maximum(u, v) * adj3).reshape(BR, N)
        agg = jnp.dot(p, faug_s[...],
                      preferred_element_type=jnp.float32)
        z = agg[:, 0:CLASSES] / agg[:, CLASSES:CLASSES + 1]
        m2 = jnp.max(z, axis=1, keepdims=True)
        lse = m2 + jnp.log(jnp.sum(jnp.exp(z - m2), axis=1, keepdims=True))
        out_ref[...] = z - lse


def kernel(x, adj, W1, a1_src, a1_dst, W2, a2_src, a2_dst):
    W1r = jnp.transpose(W1, (1, 0, 2)).reshape(INS, HD)
    eye = jnp.eye(H1, dtype=jnp.float32)
    Asrc = (eye[:, None, :] * a1_src[:, :, None]).reshape(HD, H1)
    Adst = (eye[:, None, :] * a1_dst[:, :, None]).reshape(HD, H1)
    W2r = W2[0]
    vs = jnp.dot(W2r, a2_src[0])[:, None]
    vd = jnp.dot(W2r, a2_dst[0])[:, None]

    out = pl.pallas_call(
        _fused_kernel,
        grid=(2 * NB2,),
        in_specs=[
            pl.BlockSpec((N, INS), lambda t: (0, 0)),
            pl.BlockSpec((INS, HD), lambda t: (0, 0)),
            pl.BlockSpec((HD, H1), lambda t: (0, 0)),
            pl.BlockSpec((HD, H1), lambda t: (0, 0)),
            pl.BlockSpec((BR, N), lambda t: (jnp.minimum(t, NB2 - 1), 0)),
            pl.BlockSpec((HD, CLASSES), lambda t: (0, 0)),
            pl.BlockSpec((HD, 1), lambda t: (0, 0)),
            pl.BlockSpec((HD, 1), lambda t: (0, 0)),
        ],
        out_specs=pl.BlockSpec(
            (BR, CLASSES), lambda t: (jnp.where(t < NB2, t, t - NB2), 0)),
        out_shape=jax.ShapeDtypeStruct((N, CLASSES), jnp.float32),
        scratch_shapes=[
            pltpu.VMEM((N, N), jnp.float32),
            pltpu.VMEM((N, HD), jnp.float32),
            pltpu.VMEM((N, FW), jnp.float32),
            pltpu.VMEM((N // 8, 8, H1), jnp.float32),
            pltpu.VMEM((N // 8, 8, H1), jnp.float32),
            pltpu.VMEM((H1, 8, N), jnp.float32),
            pltpu.VMEM((H1, 8, N), jnp.float32),
            pltpu.VMEM((N, FW2), jnp.float32),
            pltpu.VMEM((N // 8, 8, 1), jnp.float32),
            pltpu.VMEM((N // 8, 8, 1), jnp.float32),
            pltpu.VMEM((1, 8, N), jnp.float32),
            pltpu.VMEM((1, 8, N), jnp.float32),
        ],
    )(x, W1r, Asrc, Adst, adj, W2r, vs, vd)
    return out
